# baseline (speedup 1.0000x reference)
import functools

import jax
import jax.numpy as jnp
from jax import lax
from jax.experimental import pallas as pl
from jax.experimental.pallas import tpu as pltpu
from jax.experimental.pallas import tpu_sc as plsc

N = 10000
E = 320000
D = 128
O = 40
EPS = 1e-5

NC = 2
NS = 16
NW = NC * NS
EPW = E // NW
KC = 128
EPWP = 10240
NCH = EPWP // KC
NPH = 2
NCH2 = NCH // NPH
NP = 10240
RPT = NP // NS
DEGW = 8

_mesh = plsc.VectorSubcoreMesh(core_axis_name="c", subcore_axis_name="s")


@functools.partial(
    pl.kernel,
    mesh=_mesh,
    out_type=jax.ShapeDtypeStruct((2, NP), jnp.float32),
    scratch_types=[
        pltpu.VMEM((NP,), jnp.float32),
        pltpu.VMEM((EPW,), jnp.int32),
        pltpu.VMEM_SHARED((NS, NP), jnp.float32),
        pltpu.VMEM((NS * RPT,), jnp.float32),
        pltpu.VMEM((RPT,), jnp.float32),
    ],
    compiler_params=pltpu.CompilerParams(needs_layout_passes=False),
)
def _deg_sc(row_hbm, out_hbm, dloc, row_v, dsh, red_v, out_v):
    c = lax.axis_index("c")
    s = lax.axis_index("s")
    wid = s * NC + c
    pltpu.sync_copy(row_hbm.at[wid], row_v)
    zv = jnp.zeros((16,), jnp.float32)

    def zbody(i, carry):
        dloc[pl.ds(i * 16, 16)] = zv
        return carry

    lax.fori_loop(0, NP // 16, zbody, 0)

    ones = jnp.ones((16,), jnp.float32)

    def hbody(j, carry):
        for u in range(5):
            plsc.addupdate_scatter(dloc, [row_v[pl.ds((j * 5 + u) * 16, 16)]],
                                   ones)
        return carry

    lax.fori_loop(0, EPW // (16 * 5), hbody, 0)
    pltpu.sync_copy(dloc, dsh.at[s])
    plsc.subcore_barrier()

    for t in range(NS):
        pltpu.sync_copy(dsh.at[t, pl.ds(s * RPT, RPT)],
                        red_v.at[pl.ds(t * RPT, RPT)])

    def rbody(i, carry):
        acc = red_v[pl.ds(i * 16, 16)]
        for t in range(1, NS):
            acc = acc + red_v[pl.ds(t * RPT + i * 16, 16)]
        out_v[pl.ds(i * 16, 16)] = acc
        return carry

    lax.fori_loop(0, RPT // 16, rbody, 0)
    pltpu.sync_copy(out_v, out_hbm.at[c, pl.ds(s * RPT, RPT)])


@functools.partial(
    pl.kernel,
    mesh=_mesh,
    out_type=jax.ShapeDtypeStruct((2 * NP, D), jnp.float32),
    scratch_types=[
        pltpu.VMEM_SHARED((NP, D), jnp.float32),
        pltpu.VMEM((NCH2, KC), jnp.int32),
        pltpu.VMEM((NCH2, KC), jnp.int32),
        pltpu.VMEM((2, KC, D), jnp.float32),
        pltpu.SemaphoreType.DMA,
        pltpu.SemaphoreType.DMA,
    ],
)
def _prop_sc(h_hbm, row_hbm, col_hbm, zero_hbm, out_hbm,
             acc, row_v, col_v, msg_v, gsem, ssem):
    c = lax.axis_index("c")
    s = lax.axis_index("s")
    wid = s * NC + c
    pltpu.sync_copy(zero_hbm, acc.at[pl.ds(s * RPT, RPT)])
    plsc.subcore_barrier()

    for ph in range(NPH):
        pltpu.sync_copy(row_hbm.at[wid * NPH + ph], row_v)
        pltpu.sync_copy(col_hbm.at[wid * NPH + ph], col_v)
        pltpu.async_copy(h_hbm.at[row_v.at[0]], msg_v.at[0], gsem)

        def body(j, carry):
            p = lax.rem(j, 2)
            pltpu.make_async_copy(h_hbm.at[row_v.at[j]], msg_v.at[p],
                                  gsem).wait()

            @pl.when(j >= 1)
            def _():
                pltpu.make_async_copy(msg_v.at[1 - p],
                                      acc.at[col_v.at[j - 1]], ssem).wait()

            @pl.when(j + 1 < NCH2)
            def _():
                pltpu.async_copy(h_hbm.at[row_v.at[j + 1]], msg_v.at[1 - p],
                                 gsem)

            pltpu.async_copy(msg_v.at[p], acc.at[col_v.at[j]], ssem,
                             add=True)
            return carry

        lax.fori_loop(0, NCH2, body, 0)
        pltpu.make_async_copy(msg_v.at[(NCH2 - 1) % 2],
                              acc.at[col_v.at[NCH2 - 1]], ssem).wait()
    plsc.subcore_barrier()
    pltpu.sync_copy(acc.at[pl.ds(s * RPT, RPT)],
                    out_hbm.at[pl.ds(c * NP + s * RPT, RPT)])


BR = 2000


def _dinv_bcast(d0, d1, rows):
    deg = d0 + d1
    dinv = jnp.where(deg > 0, lax.rsqrt(deg), 0.0)
    return jnp.broadcast_to(dinv, (rows, D))


def _scale0_body(d0_ref, d1_ref, x_ref, xs_ref):
    xs_ref[...] = x_ref[...] * _dinv_bcast(d0_ref[...], d1_ref[...], BR)


def _comb_body(p_ref, d0_ref, d1_ref, o_ref):
    deg = d0_ref[...] + d1_ref[...]
    dinv2 = jnp.where(deg > 0, 1.0 / deg, 0.0)
    o_ref[...] = (p_ref[0] + p_ref[1]) * jnp.broadcast_to(dinv2, (BR, D))


def _dense1_body(p_ref, d0_ref, d1_ref, w_ref, b_ref, g_ref, be_ref, o_ref):
    d = _dinv_bcast(d0_ref[:N], d1_ref[:N], N)
    h = (p_ref[0, :N] + p_ref[1, :N]) * d
    y = jnp.dot(h, w_ref[...], preferred_element_type=jnp.float32) + b_ref[...]
    m = jnp.mean(y, axis=0, keepdims=True)
    cen = y - m
    v = jnp.mean(cen * cen, axis=0, keepdims=True)
    bn = cen * lax.rsqrt(v + EPS) * g_ref[...] + be_ref[...]
    o_ref[...] = bn * d


def _dense2_body(p_ref, d0_ref, d1_ref, w_ref, b_ref, g_ref, be_ref,
                 w2_ref, b2_ref, o_ref):
    d = _dinv_bcast(d0_ref[:N], d1_ref[:N], N)
    h = (p_ref[0, :N] + p_ref[1, :N]) * d
    y = jnp.dot(h, w_ref[...], preferred_element_type=jnp.float32) + b_ref[...]
    m = jnp.mean(y, axis=0, keepdims=True)
    cen = y - m
    v = jnp.mean(cen * cen, axis=0, keepdims=True)
    bn = cen * lax.rsqrt(v + EPS) * g_ref[...] + be_ref[...]
    o_ref[...] = (jnp.dot(bn, w2_ref[...], preferred_element_type=jnp.float32)
                  + b2_ref[...])


_scale0 = pl.pallas_call(
    _scale0_body,
    grid=(N // BR,),
    in_specs=[
        pl.BlockSpec((BR, 1), lambda i: (i, 0)),
        pl.BlockSpec((BR, 1), lambda i: (i, 0)),
        pl.BlockSpec((BR, D), lambda i: (i, 0)),
    ],
    out_specs=pl.BlockSpec((BR, D), lambda i: (i, 0)),
    out_shape=jax.ShapeDtypeStruct((N, D), jnp.float32),
)

_comb = pl.pallas_call(
    _comb_body,
    grid=(N // BR,),
    in_specs=[
        pl.BlockSpec((2, BR, D), lambda i: (0, i, 0)),
        pl.BlockSpec((BR, 1), lambda i: (i, 0)),
        pl.BlockSpec((BR, 1), lambda i: (i, 0)),
    ],
    out_specs=pl.BlockSpec((BR, D), lambda i: (i, 0)),
    out_shape=jax.ShapeDtypeStruct((N, D), jnp.float32),
)

_dense1 = pl.pallas_call(
    _dense1_body,
    out_shape=jax.ShapeDtypeStruct((N, D), jnp.float32),
)

_dense2 = pl.pallas_call(
    _dense2_body,
    out_shape=jax.ShapeDtypeStruct((N, O), jnp.float32),
)


def kernel(x, edge_index, W0, b0, g0, be0, W1, b1, g1, be1, W2, b2):
    pad = EPWP - EPW
    rowp = jnp.concatenate(
        [edge_index[0].reshape(NW, EPW),
         jnp.zeros((NW, pad), jnp.int32)], axis=1)
    colp = jnp.concatenate(
        [edge_index[1].reshape(NW, EPW),
         jnp.full((NW, pad), NP - 1, jnp.int32)], axis=1)
    row2 = rowp.reshape(NW * NPH, NCH2, KC)
    col2 = colp.reshape(NW * NPH, NCH2, KC)
    zacc = jnp.zeros((RPT, D), jnp.float32)

    rowflat = edge_index[0].reshape(NW, EPW)
    degp = _deg_sc(rowflat)
    d0 = degp[0].reshape(NP, 1)
    d1 = degp[1].reshape(NP, 1)
    hs = _scale0(d0, d1, x)

    p = _prop_sc(hs, row2, col2, zacc).reshape(2, NP, D)
    hs = _comb(p, d0, d1)
    p = _prop_sc(hs, row2, col2, zacc).reshape(2, NP, D)
    hs = _dense1(p, d0, d1, W0, b0.reshape(1, D), g0.reshape(1, D),
                 be0.reshape(1, D))

    p = _prop_sc(hs, row2, col2, zacc).reshape(2, NP, D)
    hs = _comb(p, d0, d1)
    p = _prop_sc(hs, row2, col2, zacc).reshape(2, NP, D)
    out = _dense2(p, d0, d1, W1, b1.reshape(1, D), g1.reshape(1, D),
                  be1.reshape(1, D), W2, b2.reshape(1, O))
    return out


# speedup vs baseline: 2.6089x; 2.6089x over previous
import functools

import jax
import jax.numpy as jnp
from jax import lax
from jax.experimental import pallas as pl
from jax.experimental.pallas import tpu as pltpu
from jax.experimental.pallas import tpu_sc as plsc

N = 10000
E = 320000
D = 128
O = 40
EPS = 1e-5

NC = 2
NS = 16
NW = NC * NS
EPW = E // NW
KC = 100
NCH = EPW // KC
NPH = 2
NCH2 = NCH // NPH
NP = 10240
RPT = NP // NS
DEGW = 8

_mesh = plsc.VectorSubcoreMesh(core_axis_name="c", subcore_axis_name="s")


@functools.partial(
    pl.kernel,
    mesh=_mesh,
    out_type=jax.ShapeDtypeStruct((2, NP), jnp.float32),
    scratch_types=[
        pltpu.VMEM((NP,), jnp.float32),
        pltpu.VMEM((EPW,), jnp.int32),
        pltpu.VMEM_SHARED((NS, NP), jnp.float32),
        pltpu.VMEM((NS * RPT,), jnp.float32),
        pltpu.VMEM((RPT,), jnp.float32),
    ],
    compiler_params=pltpu.CompilerParams(needs_layout_passes=False),
)
def _deg_sc(row_hbm, out_hbm, dloc, row_v, dsh, red_v, out_v):
    c = lax.axis_index("c")
    s = lax.axis_index("s")
    wid = s * NC + c
    pltpu.sync_copy(row_hbm.at[wid], row_v)
    zv = jnp.zeros((16,), jnp.float32)

    def zbody(i, carry):
        dloc[pl.ds(i * 16, 16)] = zv
        return carry

    lax.fori_loop(0, NP // 16, zbody, 0)

    ones = jnp.ones((16,), jnp.float32)

    def hbody(j, carry):
        for u in range(5):
            plsc.addupdate_scatter(dloc, [row_v[pl.ds((j * 5 + u) * 16, 16)]],
                                   ones)
        return carry

    lax.fori_loop(0, EPW // (16 * 5), hbody, 0)
    pltpu.sync_copy(dloc, dsh.at[s])
    plsc.subcore_barrier()

    for t in range(NS):
        pltpu.sync_copy(dsh.at[t, pl.ds(s * RPT, RPT)],
                        red_v.at[pl.ds(t * RPT, RPT)])

    def rbody(i, carry):
        acc = red_v[pl.ds(i * 16, 16)]
        for t in range(1, NS):
            acc = acc + red_v[pl.ds(t * RPT + i * 16, 16)]
        out_v[pl.ds(i * 16, 16)] = acc
        return carry

    lax.fori_loop(0, RPT // 16, rbody, 0)
    pltpu.sync_copy(out_v, out_hbm.at[c, pl.ds(s * RPT, RPT)])


@functools.partial(
    pl.kernel,
    mesh=_mesh,
    out_type=jax.ShapeDtypeStruct((2 * NP, D), jnp.float32),
    scratch_types=[
        pltpu.VMEM_SHARED((NP, D), jnp.float32),
        pltpu.VMEM((NCH2, KC), jnp.int32),
        pltpu.VMEM((NCH2, KC), jnp.int32),
        pltpu.VMEM((2, KC, D), jnp.float32),
        pltpu.SemaphoreType.DMA,
        pltpu.SemaphoreType.DMA,
    ],
)
def _prop_sc(h_hbm, row_hbm, col_hbm, zero_hbm, out_hbm,
             acc, row_v, col_v, msg_v, gsem, ssem):
    c = lax.axis_index("c")
    s = lax.axis_index("s")
    wid = s * NC + c
    pltpu.sync_copy(zero_hbm, acc.at[pl.ds(s * RPT, RPT)])
    plsc.subcore_barrier()

    for ph in range(NPH):
        pltpu.sync_copy(row_hbm.at[wid * NPH + ph], row_v)
        pltpu.sync_copy(col_hbm.at[wid * NPH + ph], col_v)
        pltpu.async_copy(h_hbm.at[row_v.at[0]], msg_v.at[0], gsem)

        def body(j, carry):
            p = lax.rem(j, 2)
            pltpu.make_async_copy(h_hbm.at[row_v.at[j]], msg_v.at[p],
                                  gsem).wait()

            @pl.when(j >= 1)
            def _():
                pltpu.make_async_copy(msg_v.at[1 - p],
                                      acc.at[col_v.at[j - 1]], ssem).wait()

            @pl.when(j + 1 < NCH2)
            def _():
                pltpu.async_copy(h_hbm.at[row_v.at[j + 1]], msg_v.at[1 - p],
                                 gsem)

            pltpu.async_copy(msg_v.at[p], acc.at[col_v.at[j]], ssem,
                             add=True)
            return carry

        lax.fori_loop(0, NCH2, body, 0)
        pltpu.make_async_copy(msg_v.at[(NCH2 - 1) % 2],
                              acc.at[col_v.at[NCH2 - 1]], ssem).wait()
    plsc.subcore_barrier()
    pltpu.sync_copy(acc.at[pl.ds(s * RPT, RPT)],
                    out_hbm.at[pl.ds(c * NP + s * RPT, RPT)])


BR = 2000


def _scale0_body(d0_ref, d1_ref, x_ref, dvb_ref, xs_ref):
    deg = d0_ref[...] + d1_ref[...]
    dinv = jnp.where(deg > 0, lax.rsqrt(deg), 0.0)
    dvb = jnp.broadcast_to(dinv, (BR, D))
    dvb_ref[...] = dvb
    xs_ref[...] = x_ref[...] * dvb


def _comb_body(p_ref, dvb_ref, o_ref):
    d = dvb_ref[...]
    o_ref[...] = (p_ref[0] + p_ref[1]) * (d * d)


def _dense1_body(p_ref, dvb_ref, w_ref, b_ref, g_ref, be_ref, o_ref):
    d = dvb_ref[:N]
    h = (p_ref[0, :N] + p_ref[1, :N]) * d
    y = jnp.dot(h, w_ref[...], preferred_element_type=jnp.float32) + b_ref[...]
    m = jnp.mean(y, axis=0, keepdims=True)
    cen = y - m
    v = jnp.mean(cen * cen, axis=0, keepdims=True)
    bn = cen * lax.rsqrt(v + EPS) * g_ref[...] + be_ref[...]
    o_ref[...] = bn * d


def _dense2_body(p_ref, dvb_ref, w_ref, b_ref, g_ref, be_ref,
                 w2_ref, b2_ref, o_ref):
    d = dvb_ref[:N]
    h = (p_ref[0, :N] + p_ref[1, :N]) * d
    y = jnp.dot(h, w_ref[...], preferred_element_type=jnp.float32) + b_ref[...]
    m = jnp.mean(y, axis=0, keepdims=True)
    cen = y - m
    v = jnp.mean(cen * cen, axis=0, keepdims=True)
    bn = cen * lax.rsqrt(v + EPS) * g_ref[...] + be_ref[...]
    o_ref[...] = (jnp.dot(bn, w2_ref[...], preferred_element_type=jnp.float32)
                  + b2_ref[...])


_scale0 = pl.pallas_call(
    _scale0_body,
    grid=(N // BR,),
    in_specs=[
        pl.BlockSpec((BR, 1), lambda i: (i, 0)),
        pl.BlockSpec((BR, 1), lambda i: (i, 0)),
        pl.BlockSpec((BR, D), lambda i: (i, 0)),
    ],
    out_specs=[
        pl.BlockSpec((BR, D), lambda i: (i, 0)),
        pl.BlockSpec((BR, D), lambda i: (i, 0)),
    ],
    out_shape=[jax.ShapeDtypeStruct((N, D), jnp.float32)] * 2,
)

_comb = pl.pallas_call(
    _comb_body,
    grid=(N // BR,),
    in_specs=[
        pl.BlockSpec((2, BR, D), lambda i: (0, i, 0)),
        pl.BlockSpec((BR, D), lambda i: (i, 0)),
    ],
    out_specs=pl.BlockSpec((BR, D), lambda i: (i, 0)),
    out_shape=jax.ShapeDtypeStruct((N, D), jnp.float32),
)

_dense1 = pl.pallas_call(
    _dense1_body,
    out_shape=jax.ShapeDtypeStruct((N, D), jnp.float32),
)

_dense2 = pl.pallas_call(
    _dense2_body,
    out_shape=jax.ShapeDtypeStruct((N, O), jnp.float32),
)


def kernel(x, edge_index, W0, b0, g0, be0, W1, b1, g1, be1, W2, b2):
    row2 = edge_index[0].reshape(NW * NPH, NCH2, KC)
    col2 = edge_index[1].reshape(NW * NPH, NCH2, KC)
    zacc = jnp.zeros((RPT, D), jnp.float32)

    rowflat = edge_index[0].reshape(NW, EPW)
    degp = _deg_sc(rowflat)
    d0 = degp[0].reshape(NP, 1)
    d1 = degp[1].reshape(NP, 1)
    dvb, hs = _scale0(d0, d1, x)

    p = _prop_sc(hs, row2, col2, zacc).reshape(2, NP, D)
    hs = _comb(p, dvb)
    p = _prop_sc(hs, row2, col2, zacc).reshape(2, NP, D)
    hs = _dense1(p, dvb, W0, b0.reshape(1, D), g0.reshape(1, D),
                 be0.reshape(1, D))

    p = _prop_sc(hs, row2, col2, zacc).reshape(2, NP, D)
    hs = _comb(p, dvb)
    p = _prop_sc(hs, row2, col2, zacc).reshape(2, NP, D)
    out = _dense2(p, dvb, W1, b1.reshape(1, D), g1.reshape(1, D),
                  be1.reshape(1, D), W2, b2.reshape(1, O))
    return out


# speedup vs baseline: 2.6462x; 1.0143x over previous
import functools

import jax
import jax.numpy as jnp
from jax import lax
from jax.experimental import pallas as pl
from jax.experimental.pallas import tpu as pltpu
from jax.experimental.pallas import tpu_sc as plsc

N = 10000
E = 320000
D = 128
O = 40
EPS = 1e-5

NC = 2
NS = 16
NW = NC * NS
EPW = E // NW
KC = 100
NCH = EPW // KC
NPH = 2
NCH2 = NCH // NPH
NP = 10240
RPT = NP // NS
DEGW = 8

_mesh = plsc.VectorSubcoreMesh(core_axis_name="c", subcore_axis_name="s")


@functools.partial(
    pl.kernel,
    mesh=_mesh,
    out_type=jax.ShapeDtypeStruct((2, NP), jnp.float32),
    scratch_types=[
        pltpu.VMEM((NP,), jnp.float32),
        pltpu.VMEM((EPW,), jnp.int32),
        pltpu.VMEM_SHARED((NS, NP), jnp.float32),
        pltpu.VMEM((NS * RPT,), jnp.float32),
        pltpu.VMEM((RPT,), jnp.float32),
    ],
    compiler_params=pltpu.CompilerParams(needs_layout_passes=False),
)
def _deg_sc(row_hbm, out_hbm, dloc, row_v, dsh, red_v, out_v):
    c = lax.axis_index("c")
    s = lax.axis_index("s")
    wid = s * NC + c
    pltpu.sync_copy(row_hbm.at[wid], row_v)
    zv = jnp.zeros((16,), jnp.float32)

    def zbody(i, carry):
        dloc[pl.ds(i * 16, 16)] = zv
        return carry

    lax.fori_loop(0, NP // 16, zbody, 0)

    ones = jnp.ones((16,), jnp.float32)

    def hbody(j, carry):
        for u in range(5):
            plsc.addupdate_scatter(dloc, [row_v[pl.ds((j * 5 + u) * 16, 16)]],
                                   ones)
        return carry

    lax.fori_loop(0, EPW // (16 * 5), hbody, 0)
    pltpu.sync_copy(dloc, dsh.at[s])
    plsc.subcore_barrier()

    for t in range(NS):
        pltpu.sync_copy(dsh.at[t, pl.ds(s * RPT, RPT)],
                        red_v.at[pl.ds(t * RPT, RPT)])

    def rbody(i, carry):
        acc = red_v[pl.ds(i * 16, 16)]
        for t in range(1, NS):
            acc = acc + red_v[pl.ds(t * RPT + i * 16, 16)]
        out_v[pl.ds(i * 16, 16)] = acc
        return carry

    lax.fori_loop(0, RPT // 16, rbody, 0)
    pltpu.sync_copy(out_v, out_hbm.at[c, pl.ds(s * RPT, RPT)])


@functools.partial(
    pl.kernel,
    mesh=_mesh,
    out_type=jax.ShapeDtypeStruct((2 * NP, D), jnp.float32),
    scratch_types=[
        pltpu.VMEM_SHARED((NP, D), jnp.float32),
        pltpu.VMEM((NCH2, KC), jnp.int32),
        pltpu.VMEM((NCH2, KC), jnp.int32),
        pltpu.VMEM((2, KC, D), jnp.float32),
        pltpu.SemaphoreType.DMA,
        pltpu.SemaphoreType.DMA,
        pltpu.SemaphoreType.DMA,
    ],
)
def _prop_sc(h_hbm, row_hbm, col_hbm, zero_hbm, out_hbm,
             acc, row_v, col_v, msg_v, gsem, ssem, zsem):
    c = lax.axis_index("c")
    s = lax.axis_index("s")
    wid = s * NC + c
    zdesc = pltpu.async_copy(zero_hbm, acc.at[pl.ds(s * RPT, RPT)], zsem)
    pltpu.sync_copy(row_hbm.at[wid * NPH], row_v)
    pltpu.sync_copy(col_hbm.at[wid * NPH], col_v)
    pltpu.async_copy(h_hbm.at[row_v.at[0]], msg_v.at[0], gsem)
    zdesc.wait()
    plsc.subcore_barrier()

    for ph in range(NPH):
        if ph > 0:
            pltpu.sync_copy(row_hbm.at[wid * NPH + ph], row_v)
            pltpu.sync_copy(col_hbm.at[wid * NPH + ph], col_v)
            pltpu.async_copy(h_hbm.at[row_v.at[0]], msg_v.at[0], gsem)

        def body(j, carry):
            p = lax.rem(j, 2)
            pltpu.make_async_copy(h_hbm.at[row_v.at[j]], msg_v.at[p],
                                  gsem).wait()

            @pl.when(j >= 1)
            def _():
                pltpu.make_async_copy(msg_v.at[1 - p],
                                      acc.at[col_v.at[j - 1]], ssem).wait()

            @pl.when(j + 1 < NCH2)
            def _():
                pltpu.async_copy(h_hbm.at[row_v.at[j + 1]], msg_v.at[1 - p],
                                 gsem)

            pltpu.async_copy(msg_v.at[p], acc.at[col_v.at[j]], ssem,
                             add=True)
            return carry

        lax.fori_loop(0, NCH2, body, 0)
        pltpu.make_async_copy(msg_v.at[(NCH2 - 1) % 2],
                              acc.at[col_v.at[NCH2 - 1]], ssem).wait()
    plsc.subcore_barrier()
    pltpu.sync_copy(acc.at[pl.ds(s * RPT, RPT)],
                    out_hbm.at[pl.ds(c * NP + s * RPT, RPT)])


BR = 2000


def _scale0_body(d0_ref, d1_ref, x_ref, dvb_ref, xs_ref):
    deg = d0_ref[...] + d1_ref[...]
    dinv = jnp.where(deg > 0, lax.rsqrt(deg), 0.0)
    dvb = jnp.broadcast_to(dinv, (BR, D))
    dvb_ref[...] = dvb
    xs_ref[...] = x_ref[...] * dvb


def _comb_body(p_ref, dvb_ref, o_ref):
    d = dvb_ref[...]
    o_ref[...] = (p_ref[0] + p_ref[1]) * (d * d)


def _dense1_body(p_ref, dvb_ref, w_ref, b_ref, g_ref, be_ref, o_ref):
    d = dvb_ref[:N]
    h = (p_ref[0, :N] + p_ref[1, :N]) * d
    y = jnp.dot(h, w_ref[...], preferred_element_type=jnp.float32) + b_ref[...]
    m = jnp.mean(y, axis=0, keepdims=True)
    cen = y - m
    v = jnp.mean(cen * cen, axis=0, keepdims=True)
    bn = cen * lax.rsqrt(v + EPS) * g_ref[...] + be_ref[...]
    o_ref[...] = bn * d


def _dense2_body(p_ref, dvb_ref, w_ref, b_ref, g_ref, be_ref,
                 w2_ref, b2_ref, o_ref):
    d = dvb_ref[:N]
    h = (p_ref[0, :N] + p_ref[1, :N]) * d
    y = jnp.dot(h, w_ref[...], preferred_element_type=jnp.float32) + b_ref[...]
    m = jnp.mean(y, axis=0, keepdims=True)
    cen = y - m
    v = jnp.mean(cen * cen, axis=0, keepdims=True)
    bn = cen * lax.rsqrt(v + EPS) * g_ref[...] + be_ref[...]
    o_ref[...] = (jnp.dot(bn, w2_ref[...], preferred_element_type=jnp.float32)
                  + b2_ref[...])


_scale0 = pl.pallas_call(
    _scale0_body,
    grid=(N // BR,),
    in_specs=[
        pl.BlockSpec((BR, 1), lambda i: (i, 0)),
        pl.BlockSpec((BR, 1), lambda i: (i, 0)),
        pl.BlockSpec((BR, D), lambda i: (i, 0)),
    ],
    out_specs=[
        pl.BlockSpec((BR, D), lambda i: (i, 0)),
        pl.BlockSpec((BR, D), lambda i: (i, 0)),
    ],
    out_shape=[jax.ShapeDtypeStruct((N, D), jnp.float32)] * 2,
)

_comb = pl.pallas_call(
    _comb_body,
    grid=(N // BR,),
    in_specs=[
        pl.BlockSpec((2, BR, D), lambda i: (0, i, 0)),
        pl.BlockSpec((BR, D), lambda i: (i, 0)),
    ],
    out_specs=pl.BlockSpec((BR, D), lambda i: (i, 0)),
    out_shape=jax.ShapeDtypeStruct((N, D), jnp.float32),
)

_dense1 = pl.pallas_call(
    _dense1_body,
    out_shape=jax.ShapeDtypeStruct((N, D), jnp.float32),
)

_dense2 = pl.pallas_call(
    _dense2_body,
    out_shape=jax.ShapeDtypeStruct((N, O), jnp.float32),
)


def kernel(x, edge_index, W0, b0, g0, be0, W1, b1, g1, be1, W2, b2):
    row2 = edge_index[0].reshape(NW * NPH, NCH2, KC)
    col2 = edge_index[1].reshape(NW * NPH, NCH2, KC)
    zacc = jnp.zeros((RPT, D), jnp.float32)

    rowflat = edge_index[0].reshape(NW, EPW)
    degp = _deg_sc(rowflat)
    d0 = degp[0].reshape(NP, 1)
    d1 = degp[1].reshape(NP, 1)
    dvb, hs = _scale0(d0, d1, x)

    p = _prop_sc(hs, row2, col2, zacc).reshape(2, NP, D)
    hs = _comb(p, dvb)
    p = _prop_sc(hs, row2, col2, zacc).reshape(2, NP, D)
    hs = _dense1(p, dvb, W0, b0.reshape(1, D), g0.reshape(1, D),
                 be0.reshape(1, D))

    p = _prop_sc(hs, row2, col2, zacc).reshape(2, NP, D)
    hs = _comb(p, dvb)
    p = _prop_sc(hs, row2, col2, zacc).reshape(2, NP, D)
    out = _dense2(p, dvb, W1, b1.reshape(1, D), g1.reshape(1, D),
                  be1.reshape(1, D), W2, b2.reshape(1, O))
    return out
